# baseline (speedup 1.0000x reference)
import functools

import jax
import jax.numpy as jnp
from jax import lax
from jax.experimental import pallas as pl
from jax.experimental.pallas import tpu as pltpu
from jax.experimental.pallas import tpu_sc as plsc

NC, NS, LANES = 2, 16, 16
CH = 80
PIECE = 25
NACC = 10240
WR = 80


def _sc_mesh():
    return plsc.VectorSubcoreMesh(
        core_axis_name="c", subcore_axis_name="s", num_cores=NC, num_subcores=NS
    )


def _degree_hist(src3d, dst3d, n_pad):
    _, nch_w, ch = src3d.shape
    zb_len = n_pad // NS

    def body(src_hbm, dst_hbm, din_hbm, dout_hbm, idx_v, ones_v, zb_v, hist_sp):
        c = lax.axis_index("c")
        s = lax.axis_index("s")

        def zloop(i, carry):
            zb_v[pl.ds(i * LANES, LANES)] = jnp.zeros((LANES,), jnp.float32)
            return carry

        lax.fori_loop(0, zb_len // LANES, zloop, 0)
        for i in range(ch // LANES):
            ones_v[pl.ds(i * LANES, LANES)] = jnp.ones((LANES,), jnp.float32)
        pltpu.sync_copy(zb_v, hist_sp.at[pl.ds(s * zb_len, zb_len)])
        plsc.subcore_barrier()

        @pl.when(c == 0)
        def _():
            pltpu.sync_copy(dst_hbm.at[s], idx_v)

        @pl.when(c == 1)
        def _():
            pltpu.sync_copy(src_hbm.at[s], idx_v)

        def chunk(j, carry):
            pltpu.sync_copy(ones_v, hist_sp.at[idx_v.at[j]], add=True)
            return carry

        lax.fori_loop(0, nch_w, chunk, 0)
        plsc.subcore_barrier()
        pltpu.sync_copy(hist_sp.at[pl.ds(s * zb_len, zb_len)], zb_v)

        @pl.when(c == 0)
        def _():
            pltpu.sync_copy(zb_v, din_hbm.at[pl.ds(s * zb_len, zb_len)])

        @pl.when(c == 1)
        def _():
            pltpu.sync_copy(zb_v, dout_hbm.at[pl.ds(s * zb_len, zb_len)])

    call = pl.kernel(
        body,
        out_type=[
            jax.ShapeDtypeStruct((n_pad,), jnp.float32),
            jax.ShapeDtypeStruct((n_pad,), jnp.float32),
        ],
        mesh=_sc_mesh(),
        scratch_types=[
            pltpu.VMEM((nch_w, ch), jnp.int32),
            pltpu.VMEM((ch,), jnp.float32),
            pltpu.VMEM((zb_len,), jnp.float32),
            pltpu.VMEM_SHARED((n_pad,), jnp.float32),
        ],
    )
    return call(src3d, dst3d)


def _edge_aggregate(h_flat, src_flat, dst4d, n):
    e = src_flat.shape[0]
    _, npieces, piece, ch = dst4d.shape
    epw = e // NS
    eps = piece * ch
    zpt = NACC // NS
    own = n - (NS - 1) * zpt

    def body(h_hbm, src_hbm, dst_hbm, out_hbm,
             src_v, dst_v, b0, b1, sg0, sg1, ss0, ss1, acc_sp):
        c = lax.axis_index("c")
        s = lax.axis_index("s")
        bufs = (b0, b1)
        sgs = (sg0, sg1)
        sss = (ss0, ss1)
        row_off = c * n

        def gstart(jj, u):
            pltpu.async_copy(
                h_hbm.at[src_v.at[pl.ds(jj * ch, ch)]], bufs[u], sgs[u])

        def gwait(jj, u):
            pltpu.make_async_copy(
                h_hbm.at[src_v.at[pl.ds(jj * ch, ch)]], bufs[u], sgs[u]
            ).wait()

        def sstart(jj, u):
            pltpu.async_copy(
                bufs[u], acc_sp.at[dst_v.at[jj]], sss[u], add=True)

        def sdrain(u):
            pltpu.make_async_copy(
                h_hbm.at[pl.ds(0, ch)], bufs[u], sss[u]
            ).wait()

        def zrow(i, carry):
            for k in range(128 // LANES):
                b0[i, pl.ds(k * LANES, LANES)] = jnp.zeros(
                    (LANES,), jnp.float32)
            return carry

        lax.fori_loop(0, ch, zrow, 0)
        for r in range(zpt // WR):
            pltpu.sync_copy(b0, acc_sp.at[pl.ds(s * zpt + r * WR, WR)])
        plsc.subcore_barrier()

        def piece_body(q, carry):
            pltpu.sync_copy(
                src_hbm.at[pl.ds(s * epw + q * eps, eps)], src_v)
            pltpu.sync_copy(dst_hbm.at[s, q], dst_v)

            def shift(i, carry2):
                sl = pl.ds(i * LANES, LANES)
                src_v[sl] = src_v[sl] + row_off
                return carry2

            lax.fori_loop(0, eps // LANES, shift, 0)

            gstart(0, 0)

            def pairs(j2, carry2):
                for u in range(2):
                    jj = j2 * 2 + u
                    gwait(jj, u)

                    @pl.when(jj >= 1)
                    def _():
                        sdrain(1 - u)

                    @pl.when(jj + 1 < piece)
                    def _():
                        gstart(jj + 1, 1 - u)

                    sstart(jj, u)
                return carry2

            lax.fori_loop(0, piece // 2, pairs, 0)
            lastu = (piece - 1) % 2
            gwait(piece - 1, lastu)
            sdrain(1 - lastu)
            sstart(piece - 1, lastu)
            sdrain(lastu)
            return carry

        lax.fori_loop(0, npieces, piece_body, 0)
        plsc.subcore_barrier()

        nwr = jnp.where(s == NS - 1, own // WR, zpt // WR)

        def wout(r, carry):
            sl = pl.ds(s * zpt + r * WR, WR)
            pltpu.sync_copy(acc_sp.at[sl], b0)
            pltpu.sync_copy(b0, out_hbm.at[c, sl])
            return carry

        lax.fori_loop(0, nwr, wout, 0)

    call = pl.kernel(
        body,
        out_type=jax.ShapeDtypeStruct((NC, n, 128), jnp.float32),
        mesh=_sc_mesh(),
        scratch_types=[
            pltpu.VMEM((eps,), jnp.int32),
            pltpu.VMEM((piece, ch), jnp.int32),
            pltpu.VMEM((ch, 128), jnp.float32),
            pltpu.VMEM((ch, 128), jnp.float32),
            pltpu.SemaphoreType.DMA,
            pltpu.SemaphoreType.DMA,
            pltpu.SemaphoreType.DMA,
            pltpu.SemaphoreType.DMA,
            pltpu.VMEM_SHARED((NACC, 128), jnp.float32),
        ],
    )
    return call(h_flat, src_flat, dst4d)


def _rscale(deg_blk):
    return lax.rsqrt(jnp.maximum(deg_blk, 1.0))


def _tc_layer1(x, w1, dout, bn):
    n, d_in = x.shape
    d_h = w1.shape[1]
    grid = n // bn

    def body(x_ref, w_ref, do_ref, out_ref):
        rout = _rscale(do_ref[...])
        h = jnp.dot(x_ref[...], w_ref[...], preferred_element_type=jnp.float32)
        hs = h * rout
        out_ref[0] = hs[:, : d_h // 2]
        out_ref[1] = hs[:, d_h // 2:]

    return pl.pallas_call(
        body,
        grid=(grid,),
        in_specs=[
            pl.BlockSpec((bn, d_in), lambda i: (i, 0)),
            pl.BlockSpec(w1.shape, lambda i: (0, 0)),
            pl.BlockSpec((bn, 1), lambda i: (i, 0)),
        ],
        out_specs=pl.BlockSpec((2, bn, d_h // 2), lambda i: (0, i, 0)),
        out_shape=jax.ShapeDtypeStruct((2, n, d_h // 2), jnp.float32),
    )(x, w1, dout)


def _tc_layer2(a1, din, dout, b1, w2, bn):
    _, n, dh2 = a1.shape
    d_h = 2 * dh2
    grid = n // bn

    def body(a_ref, di_ref, do_ref, b_ref, w_ref, out_ref):
        rin = _rscale(di_ref[...])
        rout = _rscale(do_ref[...])
        a = jnp.concatenate([a_ref[0], a_ref[1]], axis=1)
        h = jnp.maximum(a * rin + b_ref[...], 0.0)
        h2 = jnp.dot(h, w_ref[...], preferred_element_type=jnp.float32)
        hs = h2 * rout
        out_ref[0] = hs[:, :dh2]
        out_ref[1] = hs[:, dh2:]

    return pl.pallas_call(
        body,
        grid=(grid,),
        in_specs=[
            pl.BlockSpec((2, bn, dh2), lambda i: (0, i, 0)),
            pl.BlockSpec((bn, 1), lambda i: (i, 0)),
            pl.BlockSpec((bn, 1), lambda i: (i, 0)),
            pl.BlockSpec((1, d_h), lambda i: (0, 0)),
            pl.BlockSpec((d_h, d_h), lambda i: (0, 0)),
        ],
        out_specs=pl.BlockSpec((2, bn, dh2), lambda i: (0, i, 0)),
        out_shape=jax.ShapeDtypeStruct((2, n, dh2), jnp.float32),
    )(a1, din, dout, b1, w2)


def _tc_head(a2, din, b2, wp1, bp1, pa, wp2, bp2, bn):
    _, n, dh2 = a2.shape
    d_h = 2 * dh2
    grid = n // bn

    def body(a_ref, di_ref, b2_ref, wp1_ref, bp1_ref, pa_ref, wp2_ref,
             bp2_ref, out_ref):
        i = pl.program_id(0)
        rin = _rscale(di_ref[...])
        a = jnp.concatenate([a_ref[0], a_ref[1]], axis=1)
        h = a * rin + b2_ref[...]
        q = jnp.dot(h, wp1_ref[...], preferred_element_type=jnp.float32)
        q = q + bp1_ref[...]
        q = jnp.where(q > 0, q, pa_ref[0, 0] * q)
        p = jnp.dot(q, wp2_ref[...], preferred_element_type=jnp.float32)
        p = p + bp2_ref[...]
        ph = jnp.sum(p * h, axis=1)
        pn = jnp.maximum(jnp.sqrt(jnp.sum(p * p, axis=1)), 1e-12)
        hn = jnp.maximum(jnp.sqrt(jnp.sum(h * h, axis=1)), 1e-12)
        part = jnp.sum(ph / (pn * hn))

        @pl.when(i == 0)
        def _():
            out_ref[...] = jnp.zeros((1, 1), jnp.float32)

        out_ref[...] += part

        @pl.when(i == grid - 1)
        def _():
            out_ref[...] = 4.0 - 4.0 * out_ref[...] / n

    return pl.pallas_call(
        body,
        grid=(grid,),
        in_specs=[
            pl.BlockSpec((2, bn, dh2), lambda i: (0, i, 0)),
            pl.BlockSpec((bn, 1), lambda i: (i, 0)),
            pl.BlockSpec((1, d_h), lambda i: (0, 0)),
            pl.BlockSpec((d_h, d_h), lambda i: (0, 0)),
            pl.BlockSpec((1, d_h), lambda i: (0, 0)),
            pl.BlockSpec((1, 1), lambda i: (0, 0)),
            pl.BlockSpec((d_h, d_h), lambda i: (0, 0)),
            pl.BlockSpec((1, d_h), lambda i: (0, 0)),
        ],
        out_specs=pl.BlockSpec((1, 1), lambda i: (0, 0)),
        out_shape=jax.ShapeDtypeStruct((1, 1), jnp.float32),
    )(a2, din, b2, wp1, bp1, pa, wp2, bp2)


def kernel(x, edge_index, W1, b1, W2, b2, Wp1, bp1, prelu_a, Wp2, bp2):
    n, _ = x.shape
    e = edge_index.shape[1]
    n_pad = 10240
    bn = 1000

    src_flat = edge_index[0]
    dst_flat = edge_index[1]
    src3d = src_flat.reshape(NS, e // (NS * CH), CH)
    dst3d = dst_flat.reshape(NS, e // (NS * CH), CH)
    dst4d = dst_flat.reshape(NS, e // (NS * PIECE * CH), PIECE, CH)

    din_p, dout_p = _degree_hist(src3d, dst3d, n_pad)
    din = din_p[:n].reshape(n, 1)
    dout = dout_p[:n].reshape(n, 1)

    h1s = _tc_layer1(x, W1, dout, bn)
    a1 = _edge_aggregate(h1s.reshape(2 * n, 128), src_flat, dst4d, n)
    h2s = _tc_layer2(a1, din, dout, b1.reshape(1, -1), W2, bn)
    a2 = _edge_aggregate(h2s.reshape(2 * n, 128), src_flat, dst4d, n)
    out = _tc_head(
        a2, din, b2.reshape(1, -1), Wp1, bp1.reshape(1, -1),
        jnp.reshape(prelu_a, (1, 1)), Wp2, bp2.reshape(1, -1), bn,
    )
    return out[0, 0]


# speedup vs baseline: 1.0217x; 1.0217x over previous
import functools

import jax
import jax.numpy as jnp
from jax import lax
from jax.experimental import pallas as pl
from jax.experimental.pallas import tpu as pltpu
from jax.experimental.pallas import tpu_sc as plsc

NC, NS, LANES = 2, 16, 16
CH = 80
PIECE = 25
NACC = 10240
WR = 80


def _sc_mesh():
    return plsc.VectorSubcoreMesh(
        core_axis_name="c", subcore_axis_name="s", num_cores=NC, num_subcores=NS
    )


def _degree_hist(src3d, dst3d, n_pad):
    _, nch_w, ch = src3d.shape
    zb_len = n_pad // NS

    qd = 8

    def body(src_hbm, dst_hbm, din_hbm, dout_hbm, idx_v, ones_v, zb_v,
             hist_sp, sem):
        c = lax.axis_index("c")
        s = lax.axis_index("s")

        def zloop(i, carry):
            zb_v[pl.ds(i * LANES, LANES)] = jnp.zeros((LANES,), jnp.float32)
            return carry

        lax.fori_loop(0, zb_len // LANES, zloop, 0)
        for i in range(ch // LANES):
            ones_v[pl.ds(i * LANES, LANES)] = jnp.ones((LANES,), jnp.float32)
        pltpu.sync_copy(zb_v, hist_sp.at[pl.ds(s * zb_len, zb_len)])
        plsc.subcore_barrier()

        @pl.when(c == 0)
        def _():
            pltpu.sync_copy(dst_hbm.at[s], idx_v)

        @pl.when(c == 1)
        def _():
            pltpu.sync_copy(src_hbm.at[s], idx_v)

        def drain_one():
            pltpu.make_async_copy(
                din_hbm.at[pl.ds(0, ch)], ones_v, sem).wait()

        def chunk(j, carry):
            @pl.when(j >= qd)
            def _():
                drain_one()

            pltpu.async_copy(ones_v, hist_sp.at[idx_v.at[j]], sem, add=True)
            return carry

        lax.fori_loop(0, nch_w, chunk, 0)
        for _ in range(qd):
            drain_one()
        plsc.subcore_barrier()
        pltpu.sync_copy(hist_sp.at[pl.ds(s * zb_len, zb_len)], zb_v)

        @pl.when(c == 0)
        def _():
            pltpu.sync_copy(zb_v, din_hbm.at[pl.ds(s * zb_len, zb_len)])

        @pl.when(c == 1)
        def _():
            pltpu.sync_copy(zb_v, dout_hbm.at[pl.ds(s * zb_len, zb_len)])

    call = pl.kernel(
        body,
        out_type=[
            jax.ShapeDtypeStruct((n_pad,), jnp.float32),
            jax.ShapeDtypeStruct((n_pad,), jnp.float32),
        ],
        mesh=_sc_mesh(),
        scratch_types=[
            pltpu.VMEM((nch_w, ch), jnp.int32),
            pltpu.VMEM((ch,), jnp.float32),
            pltpu.VMEM((zb_len,), jnp.float32),
            pltpu.VMEM_SHARED((n_pad,), jnp.float32),
            pltpu.SemaphoreType.DMA,
        ],
    )
    return call(src3d, dst3d)


def _edge_aggregate(h_flat, src_flat, dst4d, n):
    e = src_flat.shape[0]
    _, npieces, piece, ch = dst4d.shape
    epw = e // NS
    eps = piece * ch
    zpt = NACC // NS
    own = n - (NS - 1) * zpt

    def body(h_hbm, src_hbm, dst_hbm, out_hbm,
             src_v, dst_v, b0, b1, sg0, sg1, ss0, ss1, acc_sp):
        c = lax.axis_index("c")
        s = lax.axis_index("s")
        bufs = (b0, b1)
        sgs = (sg0, sg1)
        sss = (ss0, ss1)
        row_off = c * n

        def gstart(jj, u):
            pltpu.async_copy(
                h_hbm.at[src_v.at[pl.ds(jj * ch, ch)]], bufs[u], sgs[u])

        def gwait(jj, u):
            pltpu.make_async_copy(
                h_hbm.at[src_v.at[pl.ds(jj * ch, ch)]], bufs[u], sgs[u]
            ).wait()

        def sstart(jj, u):
            pltpu.async_copy(
                bufs[u], acc_sp.at[dst_v.at[jj]], sss[u], add=True)

        def sdrain(u):
            pltpu.make_async_copy(
                h_hbm.at[pl.ds(0, ch)], bufs[u], sss[u]
            ).wait()

        def zrow(i, carry):
            for k in range(128 // LANES):
                b0[i, pl.ds(k * LANES, LANES)] = jnp.zeros(
                    (LANES,), jnp.float32)
            return carry

        lax.fori_loop(0, ch, zrow, 0)
        for r in range(zpt // WR):
            pltpu.sync_copy(b0, acc_sp.at[pl.ds(s * zpt + r * WR, WR)])
        plsc.subcore_barrier()

        def piece_body(q, carry):
            pltpu.sync_copy(
                src_hbm.at[pl.ds(s * epw + q * eps, eps)], src_v)
            pltpu.sync_copy(dst_hbm.at[s, q], dst_v)

            def shift(i, carry2):
                sl = pl.ds(i * LANES, LANES)
                src_v[sl] = src_v[sl] + row_off
                return carry2

            lax.fori_loop(0, eps // LANES, shift, 0)

            gstart(0, 0)

            def pairs(j2, carry2):
                for u in range(2):
                    jj = j2 * 2 + u
                    gwait(jj, u)

                    @pl.when(jj >= 1)
                    def _():
                        sdrain(1 - u)

                    @pl.when(jj + 1 < piece)
                    def _():
                        gstart(jj + 1, 1 - u)

                    sstart(jj, u)
                return carry2

            lax.fori_loop(0, piece // 2, pairs, 0)
            lastu = (piece - 1) % 2
            gwait(piece - 1, lastu)
            sdrain(1 - lastu)
            sstart(piece - 1, lastu)
            sdrain(lastu)
            return carry

        lax.fori_loop(0, npieces, piece_body, 0)
        plsc.subcore_barrier()

        nwr = jnp.where(s == NS - 1, own // WR, zpt // WR)

        def wout(r, carry):
            sl = pl.ds(s * zpt + r * WR, WR)
            pltpu.sync_copy(acc_sp.at[sl], b0)
            pltpu.sync_copy(b0, out_hbm.at[c, sl])
            return carry

        lax.fori_loop(0, nwr, wout, 0)

    call = pl.kernel(
        body,
        out_type=jax.ShapeDtypeStruct((NC, n, 128), jnp.float32),
        mesh=_sc_mesh(),
        scratch_types=[
            pltpu.VMEM((eps,), jnp.int32),
            pltpu.VMEM((piece, ch), jnp.int32),
            pltpu.VMEM((ch, 128), jnp.float32),
            pltpu.VMEM((ch, 128), jnp.float32),
            pltpu.SemaphoreType.DMA,
            pltpu.SemaphoreType.DMA,
            pltpu.SemaphoreType.DMA,
            pltpu.SemaphoreType.DMA,
            pltpu.VMEM_SHARED((NACC, 128), jnp.float32),
        ],
    )
    return call(h_flat, src_flat, dst4d)


def _rscale(deg_blk):
    return lax.rsqrt(jnp.maximum(deg_blk, 1.0))


def _tc_layer1(x, w1, dout, bn):
    n, d_in = x.shape
    d_h = w1.shape[1]
    grid = n // bn

    def body(x_ref, w_ref, do_ref, out_ref):
        rout = _rscale(do_ref[...])
        h = jnp.dot(x_ref[...].astype(jnp.bfloat16),
                    w_ref[...].astype(jnp.bfloat16),
                    preferred_element_type=jnp.float32)
        hs = h * rout
        out_ref[0] = hs[:, : d_h // 2]
        out_ref[1] = hs[:, d_h // 2:]

    return pl.pallas_call(
        body,
        grid=(grid,),
        in_specs=[
            pl.BlockSpec((bn, d_in), lambda i: (i, 0)),
            pl.BlockSpec(w1.shape, lambda i: (0, 0)),
            pl.BlockSpec((bn, 1), lambda i: (i, 0)),
        ],
        out_specs=pl.BlockSpec((2, bn, d_h // 2), lambda i: (0, i, 0)),
        out_shape=jax.ShapeDtypeStruct((2, n, d_h // 2), jnp.float32),
    )(x, w1, dout)


def _tc_layer2(a1, din, dout, b1, w2, bn):
    _, n, dh2 = a1.shape
    d_h = 2 * dh2
    grid = n // bn

    def body(a_ref, di_ref, do_ref, b_ref, w_ref, out_ref):
        rin = _rscale(di_ref[...])
        rout = _rscale(do_ref[...])
        a = jnp.concatenate([a_ref[0], a_ref[1]], axis=1)
        h = jnp.maximum(a * rin + b_ref[...], 0.0)
        h2 = jnp.dot(h.astype(jnp.bfloat16), w_ref[...].astype(jnp.bfloat16),
                     preferred_element_type=jnp.float32)
        hs = h2 * rout
        out_ref[0] = hs[:, :dh2]
        out_ref[1] = hs[:, dh2:]

    return pl.pallas_call(
        body,
        grid=(grid,),
        in_specs=[
            pl.BlockSpec((2, bn, dh2), lambda i: (0, i, 0)),
            pl.BlockSpec((bn, 1), lambda i: (i, 0)),
            pl.BlockSpec((bn, 1), lambda i: (i, 0)),
            pl.BlockSpec((1, d_h), lambda i: (0, 0)),
            pl.BlockSpec((d_h, d_h), lambda i: (0, 0)),
        ],
        out_specs=pl.BlockSpec((2, bn, dh2), lambda i: (0, i, 0)),
        out_shape=jax.ShapeDtypeStruct((2, n, dh2), jnp.float32),
    )(a1, din, dout, b1, w2)


def _tc_head(a2, din, b2, wp1, bp1, pa, wp2, bp2, bn):
    _, n, dh2 = a2.shape
    d_h = 2 * dh2
    grid = n // bn

    def body(a_ref, di_ref, b2_ref, wp1_ref, bp1_ref, pa_ref, wp2_ref,
             bp2_ref, out_ref):
        i = pl.program_id(0)
        rin = _rscale(di_ref[...])
        a = jnp.concatenate([a_ref[0], a_ref[1]], axis=1)
        h = a * rin + b2_ref[...]
        q = jnp.dot(h.astype(jnp.bfloat16), wp1_ref[...].astype(jnp.bfloat16),
                    preferred_element_type=jnp.float32)
        q = q + bp1_ref[...]
        q = jnp.where(q > 0, q, pa_ref[0, 0] * q)
        p = jnp.dot(q.astype(jnp.bfloat16), wp2_ref[...].astype(jnp.bfloat16),
                    preferred_element_type=jnp.float32)
        p = p + bp2_ref[...]
        ph = jnp.sum(p * h, axis=1)
        pn = jnp.maximum(jnp.sqrt(jnp.sum(p * p, axis=1)), 1e-12)
        hn = jnp.maximum(jnp.sqrt(jnp.sum(h * h, axis=1)), 1e-12)
        part = jnp.sum(ph / (pn * hn))

        @pl.when(i == 0)
        def _():
            out_ref[...] = jnp.zeros((1, 1), jnp.float32)

        out_ref[...] += part

        @pl.when(i == grid - 1)
        def _():
            out_ref[...] = 4.0 - 4.0 * out_ref[...] / n

    return pl.pallas_call(
        body,
        grid=(grid,),
        in_specs=[
            pl.BlockSpec((2, bn, dh2), lambda i: (0, i, 0)),
            pl.BlockSpec((bn, 1), lambda i: (i, 0)),
            pl.BlockSpec((1, d_h), lambda i: (0, 0)),
            pl.BlockSpec((d_h, d_h), lambda i: (0, 0)),
            pl.BlockSpec((1, d_h), lambda i: (0, 0)),
            pl.BlockSpec((1, 1), lambda i: (0, 0)),
            pl.BlockSpec((d_h, d_h), lambda i: (0, 0)),
            pl.BlockSpec((1, d_h), lambda i: (0, 0)),
        ],
        out_specs=pl.BlockSpec((1, 1), lambda i: (0, 0)),
        out_shape=jax.ShapeDtypeStruct((1, 1), jnp.float32),
    )(a2, din, b2, wp1, bp1, pa, wp2, bp2)


def kernel(x, edge_index, W1, b1, W2, b2, Wp1, bp1, prelu_a, Wp2, bp2):
    n, _ = x.shape
    e = edge_index.shape[1]
    n_pad = 10240
    bn = 1000

    src_flat = edge_index[0]
    dst_flat = edge_index[1]
    src3d = src_flat.reshape(NS, e // (NS * CH), CH)
    dst3d = dst_flat.reshape(NS, e // (NS * CH), CH)
    dst4d = dst_flat.reshape(NS, e // (NS * PIECE * CH), PIECE, CH)

    din_p, dout_p = _degree_hist(src3d, dst3d, n_pad)
    din = din_p[:n].reshape(n, 1)
    dout = dout_p[:n].reshape(n, 1)

    h1s = _tc_layer1(x, W1, dout, bn)
    a1 = _edge_aggregate(h1s.reshape(2 * n, 128), src_flat, dst4d, n)
    h2s = _tc_layer2(a1, din, dout, b1.reshape(1, -1), W2, bn)
    a2 = _edge_aggregate(h2s.reshape(2 * n, 128), src_flat, dst4d, n)
    out = _tc_head(
        a2, din, b2.reshape(1, -1), Wp1, bp1.reshape(1, -1),
        jnp.reshape(prelu_a, (1, 1)), Wp2, bp2.reshape(1, -1), bn,
    )
    return out[0, 0]


# speedup vs baseline: 1.0388x; 1.0167x over previous
import functools

import jax
import jax.numpy as jnp
from jax import lax
from jax.experimental import pallas as pl
from jax.experimental.pallas import tpu as pltpu
from jax.experimental.pallas import tpu_sc as plsc

NC, NS, LANES = 2, 16, 16
CH = 80
PIECE = 50
NACC = 10240
WR = 80


def _sc_mesh():
    return plsc.VectorSubcoreMesh(
        core_axis_name="c", subcore_axis_name="s", num_cores=NC, num_subcores=NS
    )


def _degree_hist(src3d, dst3d, n_pad):
    _, nch_w, ch = src3d.shape
    zb_len = n_pad // NS

    qd = 8

    def body(src_hbm, dst_hbm, din_hbm, dout_hbm, idx_v, ones_v, zb_v,
             hist_sp, sem):
        c = lax.axis_index("c")
        s = lax.axis_index("s")

        def zloop(i, carry):
            zb_v[pl.ds(i * LANES, LANES)] = jnp.zeros((LANES,), jnp.float32)
            return carry

        lax.fori_loop(0, zb_len // LANES, zloop, 0)
        for i in range(ch // LANES):
            ones_v[pl.ds(i * LANES, LANES)] = jnp.ones((LANES,), jnp.float32)
        pltpu.sync_copy(zb_v, hist_sp.at[pl.ds(s * zb_len, zb_len)])
        plsc.subcore_barrier()

        @pl.when(c == 0)
        def _():
            pltpu.sync_copy(dst_hbm.at[s], idx_v)

        @pl.when(c == 1)
        def _():
            pltpu.sync_copy(src_hbm.at[s], idx_v)

        def drain_one():
            pltpu.make_async_copy(
                din_hbm.at[pl.ds(0, ch)], ones_v, sem).wait()

        def chunk(j, carry):
            @pl.when(j >= qd)
            def _():
                drain_one()

            pltpu.async_copy(ones_v, hist_sp.at[idx_v.at[j]], sem, add=True)
            return carry

        lax.fori_loop(0, nch_w, chunk, 0)
        for _ in range(qd):
            drain_one()
        plsc.subcore_barrier()
        pltpu.sync_copy(hist_sp.at[pl.ds(s * zb_len, zb_len)], zb_v)

        @pl.when(c == 0)
        def _():
            pltpu.sync_copy(zb_v, din_hbm.at[pl.ds(s * zb_len, zb_len)])

        @pl.when(c == 1)
        def _():
            pltpu.sync_copy(zb_v, dout_hbm.at[pl.ds(s * zb_len, zb_len)])

    call = pl.kernel(
        body,
        out_type=[
            jax.ShapeDtypeStruct((n_pad,), jnp.float32),
            jax.ShapeDtypeStruct((n_pad,), jnp.float32),
        ],
        mesh=_sc_mesh(),
        scratch_types=[
            pltpu.VMEM((nch_w, ch), jnp.int32),
            pltpu.VMEM((ch,), jnp.float32),
            pltpu.VMEM((zb_len,), jnp.float32),
            pltpu.VMEM_SHARED((n_pad,), jnp.float32),
            pltpu.SemaphoreType.DMA,
        ],
    )
    return call(src3d, dst3d)


def _edge_aggregate(h_flat, src_flat, dst4d, n):
    e = src_flat.shape[0]
    _, npieces, piece, ch = dst4d.shape
    epw = e // NS
    eps = piece * ch
    zpt = NACC // NS
    own = n - (NS - 1) * zpt

    def body(h_hbm, src_hbm, dst_hbm, out_hbm,
             src_v, dst_v, b0, b1, sg0, sg1, ss0, ss1, acc_sp):
        c = lax.axis_index("c")
        s = lax.axis_index("s")
        bufs = (b0, b1)
        sgs = (sg0, sg1)
        sss = (ss0, ss1)
        row_off = c * n

        def gstart(jj, u):
            pltpu.async_copy(
                h_hbm.at[src_v.at[pl.ds(jj * ch, ch)]], bufs[u], sgs[u])

        def gwait(jj, u):
            pltpu.make_async_copy(
                h_hbm.at[src_v.at[pl.ds(jj * ch, ch)]], bufs[u], sgs[u]
            ).wait()

        def sstart(jj, u):
            pltpu.async_copy(
                bufs[u], acc_sp.at[dst_v.at[jj]], sss[u], add=True)

        def sdrain(u):
            pltpu.make_async_copy(
                h_hbm.at[pl.ds(0, ch)], bufs[u], sss[u]
            ).wait()

        def zrow(i, carry):
            for k in range(128 // LANES):
                b0[i, pl.ds(k * LANES, LANES)] = jnp.zeros(
                    (LANES,), jnp.float32)
            return carry

        lax.fori_loop(0, ch, zrow, 0)
        for r in range(zpt // WR):
            pltpu.sync_copy(b0, acc_sp.at[pl.ds(s * zpt + r * WR, WR)])
        plsc.subcore_barrier()

        def piece_body(q, carry):
            pltpu.sync_copy(
                src_hbm.at[pl.ds(s * epw + q * eps, eps)], src_v)
            pltpu.sync_copy(dst_hbm.at[s, q], dst_v)

            def shift(i, carry2):
                sl = pl.ds(i * LANES, LANES)
                src_v[sl] = src_v[sl] + row_off
                return carry2

            lax.fori_loop(0, eps // LANES, shift, 0)

            gstart(0, 0)

            def pairs(j2, carry2):
                for u in range(2):
                    jj = j2 * 2 + u
                    gwait(jj, u)

                    @pl.when(jj >= 1)
                    def _():
                        sdrain(1 - u)

                    @pl.when(jj + 1 < piece)
                    def _():
                        gstart(jj + 1, 1 - u)

                    sstart(jj, u)
                return carry2

            lax.fori_loop(0, piece // 2, pairs, 0)
            lastu = (piece - 1) % 2
            if piece % 2:
                gwait(piece - 1, lastu)
                sdrain(1 - lastu)
                sstart(piece - 1, lastu)
            sdrain(lastu)
            return carry

        lax.fori_loop(0, npieces, piece_body, 0)
        plsc.subcore_barrier()

        nwr = jnp.where(s == NS - 1, own // WR, zpt // WR)

        def wout(r, carry):
            sl = pl.ds(s * zpt + r * WR, WR)
            pltpu.sync_copy(acc_sp.at[sl], b0)
            pltpu.sync_copy(b0, out_hbm.at[c, sl])
            return carry

        lax.fori_loop(0, nwr, wout, 0)

    call = pl.kernel(
        body,
        out_type=jax.ShapeDtypeStruct((NC, n, 128), jnp.float32),
        mesh=_sc_mesh(),
        scratch_types=[
            pltpu.VMEM((eps,), jnp.int32),
            pltpu.VMEM((piece, ch), jnp.int32),
            pltpu.VMEM((ch, 128), jnp.float32),
            pltpu.VMEM((ch, 128), jnp.float32),
            pltpu.SemaphoreType.DMA,
            pltpu.SemaphoreType.DMA,
            pltpu.SemaphoreType.DMA,
            pltpu.SemaphoreType.DMA,
            pltpu.VMEM_SHARED((NACC, 128), jnp.float32),
        ],
    )
    return call(h_flat, src_flat, dst4d)


def _rscale(deg_blk):
    return lax.rsqrt(jnp.maximum(deg_blk, 1.0))


def _tc_layer1(x, w1, dout, bn):
    n, d_in = x.shape
    d_h = w1.shape[1]
    grid = n // bn

    def body(x_ref, w_ref, do_ref, out_ref):
        rout = _rscale(do_ref[...])
        h = jnp.dot(x_ref[...].astype(jnp.bfloat16),
                    w_ref[...].astype(jnp.bfloat16),
                    preferred_element_type=jnp.float32)
        hs = h * rout
        out_ref[0] = hs[:, : d_h // 2]
        out_ref[1] = hs[:, d_h // 2:]

    return pl.pallas_call(
        body,
        grid=(grid,),
        in_specs=[
            pl.BlockSpec((bn, d_in), lambda i: (i, 0)),
            pl.BlockSpec(w1.shape, lambda i: (0, 0)),
            pl.BlockSpec((bn, 1), lambda i: (i, 0)),
        ],
        out_specs=pl.BlockSpec((2, bn, d_h // 2), lambda i: (0, i, 0)),
        out_shape=jax.ShapeDtypeStruct((2, n, d_h // 2), jnp.float32),
    )(x, w1, dout)


def _tc_layer2(a1, din, dout, b1, w2, bn):
    _, n, dh2 = a1.shape
    d_h = 2 * dh2
    grid = n // bn

    def body(a_ref, di_ref, do_ref, b_ref, w_ref, out_ref):
        rin = _rscale(di_ref[...])
        rout = _rscale(do_ref[...])
        a = jnp.concatenate([a_ref[0], a_ref[1]], axis=1)
        h = jnp.maximum(a * rin + b_ref[...], 0.0)
        h2 = jnp.dot(h.astype(jnp.bfloat16), w_ref[...].astype(jnp.bfloat16),
                     preferred_element_type=jnp.float32)
        hs = h2 * rout
        out_ref[0] = hs[:, :dh2]
        out_ref[1] = hs[:, dh2:]

    return pl.pallas_call(
        body,
        grid=(grid,),
        in_specs=[
            pl.BlockSpec((2, bn, dh2), lambda i: (0, i, 0)),
            pl.BlockSpec((bn, 1), lambda i: (i, 0)),
            pl.BlockSpec((bn, 1), lambda i: (i, 0)),
            pl.BlockSpec((1, d_h), lambda i: (0, 0)),
            pl.BlockSpec((d_h, d_h), lambda i: (0, 0)),
        ],
        out_specs=pl.BlockSpec((2, bn, dh2), lambda i: (0, i, 0)),
        out_shape=jax.ShapeDtypeStruct((2, n, dh2), jnp.float32),
    )(a1, din, dout, b1, w2)


def _tc_head(a2, din, b2, wp1, bp1, pa, wp2, bp2, bn):
    _, n, dh2 = a2.shape
    d_h = 2 * dh2
    grid = n // bn

    def body(a_ref, di_ref, b2_ref, wp1_ref, bp1_ref, pa_ref, wp2_ref,
             bp2_ref, out_ref):
        i = pl.program_id(0)
        rin = _rscale(di_ref[...])
        a = jnp.concatenate([a_ref[0], a_ref[1]], axis=1)
        h = a * rin + b2_ref[...]
        q = jnp.dot(h.astype(jnp.bfloat16), wp1_ref[...].astype(jnp.bfloat16),
                    preferred_element_type=jnp.float32)
        q = q + bp1_ref[...]
        q = jnp.where(q > 0, q, pa_ref[0, 0] * q)
        p = jnp.dot(q.astype(jnp.bfloat16), wp2_ref[...].astype(jnp.bfloat16),
                    preferred_element_type=jnp.float32)
        p = p + bp2_ref[...]
        ph = jnp.sum(p * h, axis=1)
        pn = jnp.maximum(jnp.sqrt(jnp.sum(p * p, axis=1)), 1e-12)
        hn = jnp.maximum(jnp.sqrt(jnp.sum(h * h, axis=1)), 1e-12)
        part = jnp.sum(ph / (pn * hn))

        @pl.when(i == 0)
        def _():
            out_ref[...] = jnp.zeros((1, 1), jnp.float32)

        out_ref[...] += part

        @pl.when(i == grid - 1)
        def _():
            out_ref[...] = 4.0 - 4.0 * out_ref[...] / n

    return pl.pallas_call(
        body,
        grid=(grid,),
        in_specs=[
            pl.BlockSpec((2, bn, dh2), lambda i: (0, i, 0)),
            pl.BlockSpec((bn, 1), lambda i: (i, 0)),
            pl.BlockSpec((1, d_h), lambda i: (0, 0)),
            pl.BlockSpec((d_h, d_h), lambda i: (0, 0)),
            pl.BlockSpec((1, d_h), lambda i: (0, 0)),
            pl.BlockSpec((1, 1), lambda i: (0, 0)),
            pl.BlockSpec((d_h, d_h), lambda i: (0, 0)),
            pl.BlockSpec((1, d_h), lambda i: (0, 0)),
        ],
        out_specs=pl.BlockSpec((1, 1), lambda i: (0, 0)),
        out_shape=jax.ShapeDtypeStruct((1, 1), jnp.float32),
    )(a2, din, b2, wp1, bp1, pa, wp2, bp2)


def kernel(x, edge_index, W1, b1, W2, b2, Wp1, bp1, prelu_a, Wp2, bp2):
    n, _ = x.shape
    e = edge_index.shape[1]
    n_pad = 10240
    bn = 1000

    src_flat = edge_index[0]
    dst_flat = edge_index[1]
    src3d = src_flat.reshape(NS, e // (NS * CH), CH)
    dst3d = dst_flat.reshape(NS, e // (NS * CH), CH)
    dst4d = dst_flat.reshape(NS, e // (NS * PIECE * CH), PIECE, CH)

    din_p, dout_p = _degree_hist(src3d, dst3d, n_pad)
    din = din_p[:n].reshape(n, 1)
    dout = dout_p[:n].reshape(n, 1)

    h1s = _tc_layer1(x, W1, dout, bn)
    a1 = _edge_aggregate(h1s.reshape(2 * n, 128), src_flat, dst4d, n)
    h2s = _tc_layer2(a1, din, dout, b1.reshape(1, -1), W2, bn)
    a2 = _edge_aggregate(h2s.reshape(2 * n, 128), src_flat, dst4d, n)
    out = _tc_head(
        a2, din, b2.reshape(1, -1), Wp1, bp1.reshape(1, -1),
        jnp.reshape(prelu_a, (1, 1)), Wp2, bp2.reshape(1, -1), bn,
    )
    return out[0, 0]


# speedup vs baseline: 1.0487x; 1.0095x over previous
import functools

import jax
import jax.numpy as jnp
from jax import lax
from jax.experimental import pallas as pl
from jax.experimental.pallas import tpu as pltpu
from jax.experimental.pallas import tpu_sc as plsc

NC, NS, LANES = 2, 16, 16
CH = 80
PIECE = 50
NACC = 10240
WR = 80


def _sc_mesh():
    return plsc.VectorSubcoreMesh(
        core_axis_name="c", subcore_axis_name="s", num_cores=NC, num_subcores=NS
    )


def _degree_hist(src3d, dst3d, n_pad):
    _, nch_w, ch = src3d.shape
    zb_len = n_pad // NS

    qd = 8

    def body(src_hbm, dst_hbm, din_hbm, dout_hbm, idx_v, ones_v, zb_v,
             hist_sp, sem):
        c = lax.axis_index("c")
        s = lax.axis_index("s")

        def zloop(i, carry):
            zb_v[pl.ds(i * LANES, LANES)] = jnp.zeros((LANES,), jnp.float32)
            return carry

        lax.fori_loop(0, zb_len // LANES, zloop, 0)
        for i in range(ch // LANES):
            ones_v[pl.ds(i * LANES, LANES)] = jnp.ones((LANES,), jnp.float32)
        pltpu.sync_copy(zb_v, hist_sp.at[pl.ds(s * zb_len, zb_len)])
        plsc.subcore_barrier()

        @pl.when(c == 0)
        def _():
            pltpu.sync_copy(dst_hbm.at[s], idx_v)

        @pl.when(c == 1)
        def _():
            pltpu.sync_copy(src_hbm.at[s], idx_v)

        def drain_one():
            pltpu.make_async_copy(
                din_hbm.at[pl.ds(0, ch)], ones_v, sem).wait()

        def chunk(j, carry):
            @pl.when(j >= qd)
            def _():
                drain_one()

            pltpu.async_copy(ones_v, hist_sp.at[idx_v.at[j]], sem, add=True)
            return carry

        lax.fori_loop(0, nch_w, chunk, 0)
        for _ in range(qd):
            drain_one()
        plsc.subcore_barrier()
        pltpu.sync_copy(hist_sp.at[pl.ds(s * zb_len, zb_len)], zb_v)

        @pl.when(c == 0)
        def _():
            pltpu.sync_copy(zb_v, din_hbm.at[pl.ds(s * zb_len, zb_len)])

        @pl.when(c == 1)
        def _():
            pltpu.sync_copy(zb_v, dout_hbm.at[pl.ds(s * zb_len, zb_len)])

    call = pl.kernel(
        body,
        out_type=[
            jax.ShapeDtypeStruct((n_pad,), jnp.float32),
            jax.ShapeDtypeStruct((n_pad,), jnp.float32),
        ],
        mesh=_sc_mesh(),
        scratch_types=[
            pltpu.VMEM((nch_w, ch), jnp.int32),
            pltpu.VMEM((ch,), jnp.float32),
            pltpu.VMEM((zb_len,), jnp.float32),
            pltpu.VMEM_SHARED((n_pad,), jnp.float32),
            pltpu.SemaphoreType.DMA,
        ],
    )
    return call(src3d, dst3d)


def _edge_aggregate(h_flat, src_flat, dst4d, n):
    e = src_flat.shape[0]
    _, npieces, piece, ch = dst4d.shape
    epw = e // NS
    eps = piece * ch
    zpt = NACC // NS
    own = n - (NS - 1) * zpt

    def body(h_hbm, src_hbm, dst_hbm, out_hbm,
             src_v, dst_v, b0, b1, sg0, sg1, ss0, ss1, acc_sp):
        c = lax.axis_index("c")
        s = lax.axis_index("s")
        bufs = (b0, b1)
        sgs = (sg0, sg1)
        sss = (ss0, ss1)
        row_off = c * n

        def gstart(jj, u):
            pltpu.async_copy(
                h_hbm.at[src_v.at[pl.ds(jj * ch, ch)]], bufs[u], sgs[u])

        def gwait(jj, u):
            pltpu.make_async_copy(
                h_hbm.at[src_v.at[pl.ds(jj * ch, ch)]], bufs[u], sgs[u]
            ).wait()

        def sstart(jj, u):
            pltpu.async_copy(
                bufs[u], acc_sp.at[dst_v.at[jj]], sss[u], add=True)

        def sdrain(u):
            pltpu.make_async_copy(
                h_hbm.at[pl.ds(0, ch)], bufs[u], sss[u]
            ).wait()

        def zrow(i, carry):
            for k in range(128 // LANES):
                b0[i, pl.ds(k * LANES, LANES)] = jnp.zeros(
                    (LANES,), jnp.float32)
            return carry

        lax.fori_loop(0, ch, zrow, 0)
        for r in range(zpt // WR):
            pltpu.sync_copy(b0, acc_sp.at[pl.ds(s * zpt + r * WR, WR)])
        plsc.subcore_barrier()

        def piece_body(q, carry):
            pltpu.sync_copy(
                src_hbm.at[pl.ds(s * epw + q * eps, eps)], src_v)
            pltpu.sync_copy(dst_hbm.at[s, q], dst_v)

            def shift(i, carry2):
                sl = pl.ds(i * LANES, LANES)
                src_v[sl] = src_v[sl] + row_off
                return carry2

            lax.fori_loop(0, eps // LANES, shift, 0)

            gstart(0, 0)

            def pairs(j2, carry2):
                for u in range(2):
                    jj = j2 * 2 + u
                    gwait(jj, u)

                    @pl.when(jj >= 1)
                    def _():
                        sdrain(1 - u)

                    @pl.when(jj + 1 < piece)
                    def _():
                        gstart(jj + 1, 1 - u)

                    sstart(jj, u)
                return carry2

            lax.fori_loop(0, piece // 2, pairs, 0)
            lastu = (piece - 1) % 2
            if piece % 2:
                gwait(piece - 1, lastu)
                sdrain(1 - lastu)
                sstart(piece - 1, lastu)
            sdrain(lastu)
            return carry

        lax.fori_loop(0, npieces, piece_body, 0)
        plsc.subcore_barrier()

        nwr = jnp.where(s == NS - 1, own // WR, zpt // WR)

        def wout(r, carry):
            sl = pl.ds(s * zpt + r * WR, WR)
            pltpu.sync_copy(acc_sp.at[sl], b0)
            pltpu.sync_copy(b0, out_hbm.at[c, sl])
            return carry

        lax.fori_loop(0, nwr, wout, 0)

    call = pl.kernel(
        body,
        out_type=jax.ShapeDtypeStruct((NC, n, 128), jnp.float32),
        mesh=_sc_mesh(),
        scratch_types=[
            pltpu.VMEM((eps,), jnp.int32),
            pltpu.VMEM((piece, ch), jnp.int32),
            pltpu.VMEM((ch, 128), jnp.float32),
            pltpu.VMEM((ch, 128), jnp.float32),
            pltpu.SemaphoreType.DMA,
            pltpu.SemaphoreType.DMA,
            pltpu.SemaphoreType.DMA,
            pltpu.SemaphoreType.DMA,
            pltpu.VMEM_SHARED((NACC, 128), jnp.float32),
        ],
    )
    return call(h_flat, src_flat, dst4d)


def _rscale(deg_blk):
    return lax.rsqrt(jnp.maximum(deg_blk, 1.0))


def _tc_layer1(x, w1, dout, bn):
    n, d_in = x.shape
    d_h = w1.shape[1]
    grid = n // bn

    def body(x_ref, w_ref, do_ref, out_ref):
        rout = _rscale(do_ref[...])
        h = jnp.dot(x_ref[...].astype(jnp.bfloat16),
                    w_ref[...].astype(jnp.bfloat16),
                    preferred_element_type=jnp.float32)
        hs = h * rout
        out_ref[0] = hs[:, : d_h // 2]
        out_ref[1] = hs[:, d_h // 2:]

    return pl.pallas_call(
        body,
        grid=(grid,),
        in_specs=[
            pl.BlockSpec((bn, d_in), lambda i: (i, 0)),
            pl.BlockSpec(w1.shape, lambda i: (0, 0)),
            pl.BlockSpec((bn, 1), lambda i: (i, 0)),
        ],
        out_specs=pl.BlockSpec((2, bn, d_h // 2), lambda i: (0, i, 0)),
        out_shape=jax.ShapeDtypeStruct((2, n, d_h // 2), jnp.float32),
    )(x, w1, dout)


def _tc_layer2(a1, din, dout, b1, w2, bn):
    _, n, dh2 = a1.shape
    d_h = 2 * dh2
    grid = n // bn

    def body(a_ref, di_ref, do_ref, b_ref, w_ref, out_ref):
        rin = _rscale(di_ref[...])
        rout = _rscale(do_ref[...])
        a = jnp.concatenate([a_ref[0], a_ref[1]], axis=1)
        h = jnp.maximum(a * rin + b_ref[...], 0.0)
        h2 = jnp.dot(h.astype(jnp.bfloat16), w_ref[...].astype(jnp.bfloat16),
                     preferred_element_type=jnp.float32)
        hs = h2 * rout
        out_ref[0] = hs[:, :dh2]
        out_ref[1] = hs[:, dh2:]

    return pl.pallas_call(
        body,
        grid=(grid,),
        in_specs=[
            pl.BlockSpec((2, bn, dh2), lambda i: (0, i, 0)),
            pl.BlockSpec((bn, 1), lambda i: (i, 0)),
            pl.BlockSpec((bn, 1), lambda i: (i, 0)),
            pl.BlockSpec((1, d_h), lambda i: (0, 0)),
            pl.BlockSpec((d_h, d_h), lambda i: (0, 0)),
        ],
        out_specs=pl.BlockSpec((2, bn, dh2), lambda i: (0, i, 0)),
        out_shape=jax.ShapeDtypeStruct((2, n, dh2), jnp.float32),
    )(a1, din, dout, b1, w2)


def _tc_head(a2, din, b2, wp1, bp1, pa, wp2, bp2, bn):
    _, n, dh2 = a2.shape
    d_h = 2 * dh2
    grid = n // bn

    def body(a_ref, di_ref, b2_ref, wp1_ref, bp1_ref, pa_ref, wp2_ref,
             bp2_ref, out_ref):
        i = pl.program_id(0)
        rin = _rscale(di_ref[...])
        a = jnp.concatenate([a_ref[0], a_ref[1]], axis=1)
        h = a * rin + b2_ref[...]
        q = jnp.dot(h.astype(jnp.bfloat16), wp1_ref[...].astype(jnp.bfloat16),
                    preferred_element_type=jnp.float32)
        q = q + bp1_ref[...]
        q = jnp.where(q > 0, q, pa_ref[0, 0] * q)
        p = jnp.dot(q.astype(jnp.bfloat16), wp2_ref[...].astype(jnp.bfloat16),
                    preferred_element_type=jnp.float32)
        p = p + bp2_ref[...]
        ph = jnp.sum(p * h, axis=1)
        pn = jnp.maximum(jnp.sqrt(jnp.sum(p * p, axis=1)), 1e-12)
        hn = jnp.maximum(jnp.sqrt(jnp.sum(h * h, axis=1)), 1e-12)
        part = jnp.sum(ph / (pn * hn))

        @pl.when(i == 0)
        def _():
            out_ref[...] = jnp.zeros((1, 1), jnp.float32)

        out_ref[...] += part

        @pl.when(i == grid - 1)
        def _():
            out_ref[...] = 4.0 - 4.0 * out_ref[...] / n

    return pl.pallas_call(
        body,
        grid=(grid,),
        in_specs=[
            pl.BlockSpec((2, bn, dh2), lambda i: (0, i, 0)),
            pl.BlockSpec((bn, 1), lambda i: (i, 0)),
            pl.BlockSpec((1, d_h), lambda i: (0, 0)),
            pl.BlockSpec((d_h, d_h), lambda i: (0, 0)),
            pl.BlockSpec((1, d_h), lambda i: (0, 0)),
            pl.BlockSpec((1, 1), lambda i: (0, 0)),
            pl.BlockSpec((d_h, d_h), lambda i: (0, 0)),
            pl.BlockSpec((1, d_h), lambda i: (0, 0)),
        ],
        out_specs=pl.BlockSpec((1, 1), lambda i: (0, 0)),
        out_shape=jax.ShapeDtypeStruct((1, 1), jnp.float32),
    )(a2, din, b2, wp1, bp1, pa, wp2, bp2)


def kernel(x, edge_index, W1, b1, W2, b2, Wp1, bp1, prelu_a, Wp2, bp2):
    n, _ = x.shape
    e = edge_index.shape[1]
    n_pad = 10240
    bn = 2000

    src_flat = edge_index[0]
    dst_flat = edge_index[1]
    src3d = src_flat.reshape(NS, e // (NS * CH), CH)
    dst3d = dst_flat.reshape(NS, e // (NS * CH), CH)
    dst4d = dst_flat.reshape(NS, e // (NS * PIECE * CH), PIECE, CH)

    din_p, dout_p = _degree_hist(src3d, dst3d, n_pad)
    din = din_p[:n].reshape(n, 1)
    dout = dout_p[:n].reshape(n, 1)

    h1s = _tc_layer1(x, W1, dout, bn)
    a1 = _edge_aggregate(h1s.reshape(2 * n, 128), src_flat, dst4d, n)
    h2s = _tc_layer2(a1, din, dout, b1.reshape(1, -1), W2, bn)
    a2 = _edge_aggregate(h2s.reshape(2 * n, 128), src_flat, dst4d, n)
    out = _tc_head(
        a2, din, b2.reshape(1, -1), Wp1, bp1.reshape(1, -1),
        jnp.reshape(prelu_a, (1, 1)), Wp2, bp2.reshape(1, -1), bn,
    )
    return out[0, 0]
